# baseline (speedup 1.0000x reference)
import functools

import jax
import jax.numpy as jnp
from jax import lax
from jax.experimental import pallas as pl
from jax.experimental.pallas import tpu as pltpu
from jax.experimental.pallas import tpu_sc as plsc

_LN_EPS = 1e-5


def _ln_body(x_ref, g_ref, b_ref, o_ref):
    x = x_ref[...]
    mean = jnp.mean(x, axis=0, keepdims=True)
    var = jnp.mean((x - mean) ** 2, axis=0, keepdims=True)
    y = (x - mean) / jnp.sqrt(var + _LN_EPS) * g_ref[...].T + b_ref[...].T
    o_ref[...] = jnp.concatenate([y.T, jnp.zeros_like(y).T], axis=-1)


def _ln_table(table_t, gamma, beta):
    d, v = table_t.shape
    vp = (v + 12799) // 12800 * 12800
    table_t = jnp.pad(table_t, ((0, 0), (0, vp - v)))
    rows = 12800
    grid = vp // rows
    return pl.pallas_call(
        _ln_body,
        grid=(grid,),
        in_specs=[
            pl.BlockSpec((d, rows), lambda i: (0, i)),
            pl.BlockSpec((1, d), lambda i: (0, 0)),
            pl.BlockSpec((1, d), lambda i: (0, 0)),
        ],
        out_specs=pl.BlockSpec((rows, 2 * d), lambda i: (i, 0)),
        out_shape=jax.ShapeDtypeStruct((vp, 2 * d), jnp.float32),
    )(table_t, gamma.reshape(1, d), beta.reshape(1, d))


def _make_sc_row_gather(n, d, chunk, nbuf=2):
    info = plsc.get_sparse_core_info()
    nw = info.num_cores * info.num_subcores
    n_per_w = n // nw
    steps = n_per_w // chunk
    assert n_per_w % chunk == 0 and steps % nbuf == 0
    mesh = plsc.VectorSubcoreMesh(core_axis_name="c", subcore_axis_name="s")

    @functools.partial(
        pl.kernel,
        out_type=jax.ShapeDtypeStruct((n, d), jnp.float32),
        mesh=mesh,
        scratch_types=[
            pltpu.VMEM((n_per_w,), jnp.int32),
        ] + [pltpu.VMEM((chunk, d), jnp.float32)] * nbuf
          + [pltpu.SemaphoreType.DMA] * (2 * nbuf),
    )
    def sc_gather(idx_hbm, table_hbm, out_hbm, idx_all, *bufs):
        rows = bufs[:nbuf]
        sem_g = bufs[nbuf:2 * nbuf]
        sem_w = bufs[2 * nbuf:3 * nbuf]
        wid = lax.axis_index("s") * info.num_cores + lax.axis_index("c")
        w_base = wid * n_per_w

        pltpu.sync_copy(idx_hbm.at[pl.ds(w_base, n_per_w)], idx_all)

        def gather(g, s):
            return pltpu.make_async_copy(
                table_hbm.at[idx_all.at[pl.ds(g * chunk, chunk)]],
                rows[s], sem_g[s])

        def write(g, s):
            return pltpu.make_async_copy(
                rows[s], out_hbm.at[pl.ds(w_base + g * chunk, chunk)],
                sem_w[s])

        for s in range(nbuf):
            gather(s, s).start()

        def body(big, carry):
            for s in range(nbuf):
                g = big * nbuf + s
                gather(g, s).wait()
                write(g, s).start()
                nxt = g + nbuf

                @pl.when(nxt < steps)
                def _():
                    write(g, s).wait()
                    gather(nxt, s).start()
            return carry

        lax.fori_loop(0, steps // nbuf, body, 0, unroll=False)
        for s in range(nbuf):
            write(steps - nbuf + s, s).wait()

    return sc_gather


def _transpose_body(gf_ref, gn_ref, o_ref):
    dn = o_ref.shape[1] - gf_ref.shape[2]
    for i in range(gf_ref.shape[1]):
        o_ref[i, : gf_ref.shape[2]] = gf_ref[:, i, :].T
        o_ref[i, gf_ref.shape[2]:] = gn_ref[:, i, :dn].T


def _transpose_body2(gf_ref, gn_ref, t_ref, o_ref):
    del t_ref
    _transpose_body(gf_ref, gn_ref, o_ref)


def _tc_transpose(g3f, g3n, d, l_total, l_off, t_partial=None):
    b, lh, df = g3f.shape
    dnp = g3n.shape[2]
    bb, lb = 128, 8
    lo = l_off // lb
    out_shape = jax.ShapeDtypeStruct((l_total, d, b), jnp.float32)
    in_specs = [
        pl.BlockSpec((bb, lb, df), lambda i, j: (j, i, 0)),
        pl.BlockSpec((bb, lb, dnp), lambda i, j: (j, i, 0)),
    ]
    out_specs = pl.BlockSpec((lb, d, bb), lambda i, j: (i + lo, 0, j))
    grid = (lh // lb, b // bb)
    if t_partial is None:
        return pl.pallas_call(
            _transpose_body, grid=grid, in_specs=in_specs,
            out_specs=out_specs, out_shape=out_shape,
        )(g3f, g3n)
    return pl.pallas_call(
        _transpose_body2, grid=grid,
        in_specs=in_specs + [pl.BlockSpec(memory_space=pl.ANY)],
        out_specs=out_specs, out_shape=out_shape,
        input_output_aliases={2: 0},
    )(g3f, g3n, t_partial)


def kernel(item_ids, frozen_emb, item_table, ln_gamma, ln_beta):
    b, l = item_ids.shape
    v, df = frozen_emb.shape
    dn = item_table.shape[1]
    d = df + dn

    normed = _ln_table(item_table.T, ln_gamma, ln_beta)
    dnp = normed.shape[1]
    nsplit = 8
    base = l // nsplit // 8 * 8
    splits = [base] * (nsplit - 1) + [l - base * (nsplit - 1)]
    t, off = None, 0
    for lh in splits:
        idsh = item_ids[:, off:off + lh].reshape(b * lh).astype(jnp.int32)
        gf = _make_sc_row_gather(b * lh, df, chunk=64)(idsh, frozen_emb)
        gn = _make_sc_row_gather(b * lh, dnp, chunk=128)(idsh, normed)
        t = _tc_transpose(gf.reshape(b, lh, df), gn.reshape(b, lh, dnp),
                          d, l, off, t)
        off += lh
    return jnp.transpose(t, (2, 0, 1))


# speedup vs baseline: 1.0080x; 1.0080x over previous
import functools

import jax
import jax.numpy as jnp
from jax import lax
from jax.experimental import pallas as pl
from jax.experimental.pallas import tpu as pltpu
from jax.experimental.pallas import tpu_sc as plsc

_LN_EPS = 1e-5


def _ln_body(x_ref, g_ref, b_ref, o_ref):
    x = x_ref[...]
    mean = jnp.mean(x, axis=0, keepdims=True)
    var = jnp.mean((x - mean) ** 2, axis=0, keepdims=True)
    y = (x - mean) / jnp.sqrt(var + _LN_EPS) * g_ref[...].T + b_ref[...].T
    o_ref[...] = jnp.concatenate([y.T, jnp.zeros_like(y).T], axis=-1)


def _ln_table(table_t, gamma, beta):
    d, v = table_t.shape
    vp = (v + 12799) // 12800 * 12800
    table_t = jnp.pad(table_t, ((0, 0), (0, vp - v)))
    rows = 12800
    grid = vp // rows
    return pl.pallas_call(
        _ln_body,
        grid=(grid,),
        in_specs=[
            pl.BlockSpec((d, rows), lambda i: (0, i)),
            pl.BlockSpec((1, d), lambda i: (0, 0)),
            pl.BlockSpec((1, d), lambda i: (0, 0)),
        ],
        out_specs=pl.BlockSpec((rows, 2 * d), lambda i: (i, 0)),
        out_shape=jax.ShapeDtypeStruct((vp, 2 * d), jnp.float32),
    )(table_t, gamma.reshape(1, d), beta.reshape(1, d))


def _make_sc_row_gather(n, d, chunk, nbuf=2):
    info = plsc.get_sparse_core_info()
    nw = info.num_cores * info.num_subcores
    n_per_w = n // nw
    steps = n_per_w // chunk
    assert n_per_w % chunk == 0 and steps % nbuf == 0
    mesh = plsc.VectorSubcoreMesh(core_axis_name="c", subcore_axis_name="s")

    @functools.partial(
        pl.kernel,
        out_type=jax.ShapeDtypeStruct((n, d), jnp.float32),
        mesh=mesh,
        scratch_types=[
            pltpu.VMEM((n_per_w,), jnp.int32),
        ] + [pltpu.VMEM((chunk, d), jnp.float32)] * nbuf
          + [pltpu.SemaphoreType.DMA] * (2 * nbuf),
    )
    def sc_gather(idx_hbm, table_hbm, out_hbm, idx_all, *bufs):
        rows = bufs[:nbuf]
        sem_g = bufs[nbuf:2 * nbuf]
        sem_w = bufs[2 * nbuf:3 * nbuf]
        wid = lax.axis_index("s") * info.num_cores + lax.axis_index("c")
        w_base = wid * n_per_w

        pltpu.sync_copy(idx_hbm.at[pl.ds(w_base, n_per_w)], idx_all)

        def gather(g, s):
            return pltpu.make_async_copy(
                table_hbm.at[idx_all.at[pl.ds(g * chunk, chunk)]],
                rows[s], sem_g[s])

        def write(g, s):
            return pltpu.make_async_copy(
                rows[s], out_hbm.at[pl.ds(w_base + g * chunk, chunk)],
                sem_w[s])

        for s in range(nbuf):
            gather(s, s).start()

        def body(big, carry):
            for s in range(nbuf):
                g = big * nbuf + s
                gather(g, s).wait()
                write(g, s).start()
                nxt = g + nbuf

                @pl.when(nxt < steps)
                def _():
                    write(g, s).wait()
                    gather(nxt, s).start()
            return carry

        lax.fori_loop(0, steps // nbuf, body, 0, unroll=False)
        for s in range(nbuf):
            write(steps - nbuf + s, s).wait()

    return sc_gather


def _transpose_body(gf_ref, gn_ref, o_ref):
    dn = o_ref.shape[1] - gf_ref.shape[2]
    for i in range(gf_ref.shape[1]):
        o_ref[i, : gf_ref.shape[2]] = gf_ref[:, i, :].T
        o_ref[i, gf_ref.shape[2]:] = gn_ref[:, i, :dn].T


def _transpose_body2(gf_ref, gn_ref, t_ref, o_ref):
    del t_ref
    _transpose_body(gf_ref, gn_ref, o_ref)


def _tc_transpose(g3f, g3n, d, l_total, l_off, t_partial=None):
    b, lh, df = g3f.shape
    dnp = g3n.shape[2]
    bb, lb = 128, 8
    lo = l_off // lb
    out_shape = jax.ShapeDtypeStruct((l_total, d, b), jnp.float32)
    in_specs = [
        pl.BlockSpec((bb, lb, df), lambda i, j: (j, i, 0)),
        pl.BlockSpec((bb, lb, dnp), lambda i, j: (j, i, 0)),
    ]
    out_specs = pl.BlockSpec((lb, d, bb), lambda i, j: (i + lo, 0, j))
    grid = (lh // lb, b // bb)
    if t_partial is None:
        return pl.pallas_call(
            _transpose_body, grid=grid, in_specs=in_specs,
            out_specs=out_specs, out_shape=out_shape,
        )(g3f, g3n)
    return pl.pallas_call(
        _transpose_body2, grid=grid,
        in_specs=in_specs + [pl.BlockSpec(memory_space=pl.ANY)],
        out_specs=out_specs, out_shape=out_shape,
        input_output_aliases={2: 0},
    )(g3f, g3n, t_partial)


def kernel(item_ids, frozen_emb, item_table, ln_gamma, ln_beta):
    b, l = item_ids.shape
    v, df = frozen_emb.shape
    dn = item_table.shape[1]
    d = df + dn

    normed = _ln_table(item_table.T, ln_gamma, ln_beta)
    dnp = normed.shape[1]
    splits = [l // 5 // 8 * 8, 0, 0, l // 5 // 8 * 8]
    splits[1] = (l - 2 * splits[0]) // 2 // 8 * 8
    splits[2] = l - 2 * splits[0] - splits[1]
    t, off = None, 0
    for lh in splits:
        idsh = item_ids[:, off:off + lh].reshape(b * lh).astype(jnp.int32)
        gf = _make_sc_row_gather(b * lh, df, chunk=64)(idsh, frozen_emb)
        gn = _make_sc_row_gather(b * lh, dnp, chunk=128)(idsh, normed)
        t = _tc_transpose(gf.reshape(b, lh, df), gn.reshape(b, lh, dnp),
                          d, l, off, t)
        off += lh
    return jnp.transpose(t, (2, 0, 1))


# speedup vs baseline: 1.0136x; 1.0056x over previous
import functools

import jax
import jax.numpy as jnp
from jax import lax
from jax.experimental import pallas as pl
from jax.experimental.pallas import tpu as pltpu
from jax.experimental.pallas import tpu_sc as plsc

_LN_EPS = 1e-5


def _ln_body(x_ref, g_ref, b_ref, o_ref):
    x = x_ref[...]
    mean = jnp.mean(x, axis=0, keepdims=True)
    var = jnp.mean((x - mean) ** 2, axis=0, keepdims=True)
    y = (x - mean) / jnp.sqrt(var + _LN_EPS) * g_ref[...].T + b_ref[...].T
    o_ref[...] = jnp.concatenate([y.T, jnp.zeros_like(y).T], axis=-1)


def _ln_table(table_t, gamma, beta):
    d, v = table_t.shape
    vp = (v + 12799) // 12800 * 12800
    table_t = jnp.pad(table_t, ((0, 0), (0, vp - v)))
    rows = 12800
    grid = vp // rows
    return pl.pallas_call(
        _ln_body,
        grid=(grid,),
        in_specs=[
            pl.BlockSpec((d, rows), lambda i: (0, i)),
            pl.BlockSpec((1, d), lambda i: (0, 0)),
            pl.BlockSpec((1, d), lambda i: (0, 0)),
        ],
        out_specs=pl.BlockSpec((rows, 2 * d), lambda i: (i, 0)),
        out_shape=jax.ShapeDtypeStruct((vp, 2 * d), jnp.float32),
    )(table_t, gamma.reshape(1, d), beta.reshape(1, d))


def _make_sc_row_gather(n, d, chunk, nbuf=2):
    info = plsc.get_sparse_core_info()
    nw = info.num_cores * info.num_subcores
    n_per_w = n // nw
    steps = n_per_w // chunk
    assert n_per_w % chunk == 0 and steps % nbuf == 0
    mesh = plsc.VectorSubcoreMesh(core_axis_name="c", subcore_axis_name="s")

    @functools.partial(
        pl.kernel,
        out_type=jax.ShapeDtypeStruct((n, d), jnp.float32),
        mesh=mesh,
        scratch_types=[
            pltpu.VMEM((n_per_w,), jnp.int32),
        ] + [pltpu.VMEM((chunk, d), jnp.float32)] * nbuf
          + [pltpu.SemaphoreType.DMA] * (2 * nbuf),
    )
    def sc_gather(idx_hbm, table_hbm, out_hbm, idx_all, *bufs):
        rows = bufs[:nbuf]
        sem_g = bufs[nbuf:2 * nbuf]
        sem_w = bufs[2 * nbuf:3 * nbuf]
        wid = lax.axis_index("s") * info.num_cores + lax.axis_index("c")
        w_base = wid * n_per_w

        pltpu.sync_copy(idx_hbm.at[pl.ds(w_base, n_per_w)], idx_all)

        def gather(g, s):
            return pltpu.make_async_copy(
                table_hbm.at[idx_all.at[pl.ds(g * chunk, chunk)]],
                rows[s], sem_g[s])

        def write(g, s):
            return pltpu.make_async_copy(
                rows[s], out_hbm.at[pl.ds(w_base + g * chunk, chunk)],
                sem_w[s])

        for s in range(nbuf):
            gather(s, s).start()

        def body(big, carry):
            for s in range(nbuf):
                g = big * nbuf + s
                gather(g, s).wait()
                write(g, s).start()
                nxt = g + nbuf

                @pl.when(nxt < steps)
                def _():
                    write(g, s).wait()
                    gather(nxt, s).start()
            return carry

        lax.fori_loop(0, steps // nbuf, body, 0, unroll=False)
        for s in range(nbuf):
            write(steps - nbuf + s, s).wait()

    return sc_gather


def _transpose_body(gf_ref, gn_ref, o_ref):
    dn = o_ref.shape[1] - gf_ref.shape[2]
    for i in range(gf_ref.shape[1]):
        o_ref[i, : gf_ref.shape[2]] = gf_ref[:, i, :].T
        o_ref[i, gf_ref.shape[2]:] = gn_ref[:, i, :dn].T


def _transpose_body2(gf_ref, gn_ref, t_ref, o_ref):
    del t_ref
    _transpose_body(gf_ref, gn_ref, o_ref)


def _tc_transpose(g3f, g3n, d, l_total, l_off, t_partial=None):
    b, lh, df = g3f.shape
    dnp = g3n.shape[2]
    bb, lb = 128, 8
    lo = l_off // lb
    out_shape = jax.ShapeDtypeStruct((l_total, d, b), jnp.float32)
    in_specs = [
        pl.BlockSpec((bb, lb, df), lambda i, j: (j, i, 0)),
        pl.BlockSpec((bb, lb, dnp), lambda i, j: (j, i, 0)),
    ]
    out_specs = pl.BlockSpec((lb, d, bb), lambda i, j: (i + lo, 0, j))
    grid = (lh // lb, b // bb)
    if t_partial is None:
        return pl.pallas_call(
            _transpose_body, grid=grid, in_specs=in_specs,
            out_specs=out_specs, out_shape=out_shape,
        )(g3f, g3n)
    return pl.pallas_call(
        _transpose_body2, grid=grid,
        in_specs=in_specs + [pl.BlockSpec(memory_space=pl.ANY)],
        out_specs=out_specs, out_shape=out_shape,
        input_output_aliases={2: 0},
    )(g3f, g3n, t_partial)


def kernel(item_ids, frozen_emb, item_table, ln_gamma, ln_beta):
    b, l = item_ids.shape
    v, df = frozen_emb.shape
    dn = item_table.shape[1]
    d = df + dn

    normed = _ln_table(item_table.T, ln_gamma, ln_beta)
    dnp = normed.shape[1]
    nsplit = 4
    base = l // nsplit // 8 * 8
    splits = [base] * (nsplit - 1) + [l - base * (nsplit - 1)]
    t, off = None, 0
    for lh in splits:
        idsh = item_ids[:, off:off + lh].reshape(b * lh).astype(jnp.int32)
        gf = _make_sc_row_gather(b * lh, df, chunk=64)(idsh, frozen_emb)
        gn = _make_sc_row_gather(b * lh, dnp, chunk=128)(idsh, normed)
        t = _tc_transpose(gf.reshape(b, lh, df), gn.reshape(b, lh, dnp),
                          d, l, off, t)
        off += lh
    return jnp.transpose(t, (2, 0, 1))
